# baseline (speedup 1.0000x reference)
import functools

import numpy as np
import jax
import jax.numpy as jnp
from jax import lax
from jax.experimental import pallas as pl
from jax.experimental.pallas import tpu as pltpu


def _i32(v) -> int:
    return int(np.uint32(v).view(np.int32))


_K1 = 0
_K2 = 42
_KS2 = _i32(np.uint32(_K1) ^ np.uint32(_K2) ^ np.uint32(0x1BD11BDA))
_ROTS = ((13, 15, 26, 6), (17, 29, 16, 24))
_KS = (_i32(_K1), _i32(_K2), _KS2)
_TINY = float(np.finfo(np.float32).tiny)
_EXP_ONE = _i32(0x3F800000)
_BIG_I32 = np.iinfo(np.int32).max


def _rotl(x, r):
    return lax.shift_left(x, np.int32(r)) | lax.shift_right_logical(
        x, np.int32(32 - r))


def _threefry_bits(i):
    x0 = jnp.full(i.shape, _KS[0], jnp.int32)
    x1 = i + np.int32(_KS[1])
    for rnd in range(5):
        for r in _ROTS[rnd % 2]:
            x0 = x0 + x1
            x1 = _rotl(x1, r)
            x1 = x1 ^ x0
        x0 = x0 + np.int32(_KS[(rnd + 1) % 3])
        x1 = x1 + np.int32(_i32(np.uint32(_KS[(rnd + 2) % 3]) + np.uint32(rnd + 1)))
    return x0 ^ x1


def _sample_kernel(x_ref, idx_ref, val_ref, base_ref, *, ncols, bc,
                   nblk_inner, nblk):
    h = pl.program_id(0)
    jj = pl.program_id(1)
    j = h * np.int32(nblk_inner) + jj
    x = x_ref[...]
    rows, cols = x.shape

    @pl.when(j == 0)
    def _():
        row = lax.broadcasted_iota(jnp.int32, (rows, cols), 0)
        lane = lax.broadcasted_iota(jnp.int32, (rows, cols), 1)
        base_ref[...] = row * np.int32(ncols) + lane

    i = base_ref[...] + j * np.int32(bc)

    bits = _threefry_bits(i)
    fb = lax.shift_right_logical(bits, np.int32(9)) | np.int32(_EXP_ONE)
    f = lax.bitcast_convert_type(fb, jnp.float32) - np.float32(1.0)
    u = jnp.maximum(f, np.float32(_TINY))
    g = -jnp.log(-jnp.log(u))
    v = x + g

    lane = lax.broadcasted_iota(jnp.int32, (rows, cols), 1)

    def _reduce_update(vv):
        bm = jnp.max(vv, axis=1, keepdims=True)
        cand = jnp.where(vv == bm, lane, np.int32(_BIG_I32))
        bi = jnp.min(cand, axis=1, keepdims=True) + j * np.int32(bc)

        @pl.when(jj == 0)
        def _():
            val_ref[...] = bm
            idx_ref[...] = bi

        @pl.when(jj != 0)
        def _():
            better = bm > val_ref[...]
            val_ref[...] = jnp.where(better, bm, val_ref[...])
            idx_ref[...] = jnp.where(better, bi, idx_ref[...])

    if ncols % bc == 0:
        _reduce_update(v)
    else:
        rem = np.int32(ncols - (nblk - 1) * bc)

        @pl.when(j == np.int32(nblk - 1))
        def _():
            _reduce_update(jnp.where(lane < rem, v, -jnp.inf))

        @pl.when(j != np.int32(nblk - 1))
        def _():
            _reduce_update(v)


@functools.partial(jax.jit, static_argnames=("block_cols", "nsplit"))
def _sample(outputs, block_cols=2048, nsplit=1):
    rows, ncols = outputs.shape
    nblk = pl.cdiv(ncols, block_cols)
    assert nblk % nsplit == 0, (nblk, nsplit)
    nblk_inner = nblk // nsplit
    idx, val = pl.pallas_call(
        functools.partial(_sample_kernel, ncols=ncols, bc=block_cols,
                          nblk_inner=nblk_inner, nblk=nblk),
        grid=(nsplit, nblk_inner),
        in_specs=[pl.BlockSpec((rows, block_cols),
                               lambda h, jj: (0, h * nblk_inner + jj))],
        out_specs=[
            pl.BlockSpec((rows, 1), lambda h, jj: (h, 0)),
            pl.BlockSpec((rows, 1), lambda h, jj: (h, 0)),
        ],
        out_shape=[
            jax.ShapeDtypeStruct((nsplit * rows, 1), jnp.int32),
            jax.ShapeDtypeStruct((nsplit * rows, 1), jnp.float32),
        ],
        scratch_shapes=[pltpu.VMEM((rows, block_cols), jnp.int32)],
        compiler_params=pltpu.CompilerParams(
            dimension_semantics=("parallel", "arbitrary")),
    )(outputs)
    idx = idx.reshape(nsplit, rows)
    val = val.reshape(nsplit, rows)
    best_v = val[0]
    best_i = idx[0]
    for s in range(1, nsplit):
        better = val[s] > best_v
        best_v = jnp.where(better, val[s], best_v)
        best_i = jnp.where(better, idx[s], best_i)
    return best_i[:, None]


def kernel(outputs):
    return _sample(outputs)


# speedup vs baseline: 1.0048x; 1.0048x over previous
import functools

import numpy as np
import jax
import jax.numpy as jnp
from jax import lax
from jax.experimental import pallas as pl
from jax.experimental.pallas import tpu as pltpu


def _i32(v) -> int:
    return int(np.uint32(v).view(np.int32))


_K1 = 0
_K2 = 42
_KS2 = _i32(np.uint32(_K1) ^ np.uint32(_K2) ^ np.uint32(0x1BD11BDA))
_ROTS = ((13, 15, 26, 6), (17, 29, 16, 24))
_KS = (_i32(_K1), _i32(_K2), _KS2)
_TINY = float(np.finfo(np.float32).tiny)
_EXP_ONE = _i32(0x3F800000)
_BIG_I32 = np.iinfo(np.int32).max


def _rotl(x, r):
    return lax.shift_left(x, np.int32(r)) | lax.shift_right_logical(
        x, np.int32(32 - r))


def _threefry_bits(i):
    x0 = jnp.full(i.shape, _KS[0], jnp.int32)
    x1 = i + np.int32(_KS[1])
    for rnd in range(5):
        for r in _ROTS[rnd % 2]:
            x0 = x0 + x1
            x1 = _rotl(x1, r)
            x1 = x1 ^ x0
        x0 = x0 + np.int32(_KS[(rnd + 1) % 3])
        x1 = x1 + np.int32(_i32(np.uint32(_KS[(rnd + 2) % 3]) + np.uint32(rnd + 1)))
    return x0 ^ x1


def _sample_kernel(x_ref, idx_ref, val_ref, *, ncols, bc,
                   nblk_inner, nblk):
    h = pl.program_id(0)
    jj = pl.program_id(1)
    j = h * np.int32(nblk_inner) + jj
    x = x_ref[...]
    rows, cols = x.shape

    row = lax.broadcasted_iota(jnp.int32, (rows, cols), 0)
    lane = lax.broadcasted_iota(jnp.int32, (rows, cols), 1)
    i = row * np.int32(ncols) + lane + j * np.int32(bc)

    bits = _threefry_bits(i)
    fb = lax.shift_right_logical(bits, np.int32(9)) | np.int32(_EXP_ONE)
    f = lax.bitcast_convert_type(fb, jnp.float32) - np.float32(1.0)
    u = jnp.maximum(f, np.float32(_TINY))
    g = -jnp.log(-jnp.log(u))
    v = x + g

    def _reduce_update(vv):
        bm = jnp.max(vv, axis=1, keepdims=True)
        cand = jnp.where(vv == bm, lane, np.int32(_BIG_I32))
        bi = jnp.min(cand, axis=1, keepdims=True) + j * np.int32(bc)

        @pl.when(jj == 0)
        def _():
            val_ref[...] = bm
            idx_ref[...] = bi

        @pl.when(jj != 0)
        def _():
            better = bm > val_ref[...]
            val_ref[...] = jnp.where(better, bm, val_ref[...])
            idx_ref[...] = jnp.where(better, bi, idx_ref[...])

    if ncols % bc == 0:
        _reduce_update(v)
    else:
        rem = np.int32(ncols - (nblk - 1) * bc)

        @pl.when(j == np.int32(nblk - 1))
        def _():
            _reduce_update(jnp.where(lane < rem, v, -jnp.inf))

        @pl.when(j != np.int32(nblk - 1))
        def _():
            _reduce_update(v)


@functools.partial(jax.jit, static_argnames=("block_cols", "nsplit"))
def _sample(outputs, block_cols=2048, nsplit=1):
    rows, ncols = outputs.shape
    nblk = pl.cdiv(ncols, block_cols)
    assert nblk % nsplit == 0, (nblk, nsplit)
    nblk_inner = nblk // nsplit
    idx, val = pl.pallas_call(
        functools.partial(_sample_kernel, ncols=ncols, bc=block_cols,
                          nblk_inner=nblk_inner, nblk=nblk),
        grid=(nsplit, nblk_inner),
        in_specs=[pl.BlockSpec((rows, block_cols),
                               lambda h, jj: (0, h * nblk_inner + jj))],
        out_specs=[
            pl.BlockSpec((rows, 1), lambda h, jj: (h, 0)),
            pl.BlockSpec((rows, 1), lambda h, jj: (h, 0)),
        ],
        out_shape=[
            jax.ShapeDtypeStruct((nsplit * rows, 1), jnp.int32),
            jax.ShapeDtypeStruct((nsplit * rows, 1), jnp.float32),
        ],
        compiler_params=pltpu.CompilerParams(
            dimension_semantics=("parallel", "arbitrary")),
    )(outputs)
    idx = idx.reshape(nsplit, rows)
    val = val.reshape(nsplit, rows)
    best_v = val[0]
    best_i = idx[0]
    for s in range(1, nsplit):
        better = val[s] > best_v
        best_v = jnp.where(better, val[s], best_v)
        best_i = jnp.where(better, idx[s], best_i)
    return best_i[:, None]


def kernel(outputs):
    return _sample(outputs)


# speedup vs baseline: 1.6569x; 1.6489x over previous
import functools

import numpy as np
import jax
import jax.numpy as jnp
from jax import lax
from jax.experimental import pallas as pl


def _i32(v) -> int:
    return int(np.uint32(v).view(np.int32))


_K1 = 0
_K2 = 42
_KS2 = _i32(np.uint32(_K1) ^ np.uint32(_K2) ^ np.uint32(0x1BD11BDA))
_ROTS = ((13, 15, 26, 6), (17, 29, 16, 24))
_KS = (_i32(_K1), _i32(_K2), _KS2)
_TINY = float(np.finfo(np.float32).tiny)
_EXP_ONE = _i32(0x3F800000)
_BIG_I32 = np.iinfo(np.int32).max


def _rotl(x, r):
    return lax.shift_left(x, np.int32(r)) | lax.shift_right_logical(
        x, np.int32(32 - r))


def _threefry_bits(i):
    x0 = jnp.full(i.shape, _KS[0], jnp.int32)
    x1 = i + np.int32(_KS[1])
    for rnd in range(5):
        for r in _ROTS[rnd % 2]:
            x0 = x0 + x1
            x1 = _rotl(x1, r)
            x1 = x1 ^ x0
        x0 = x0 + np.int32(_KS[(rnd + 1) % 3])
        x1 = x1 + np.int32(_i32(np.uint32(_KS[(rnd + 2) % 3]) + np.uint32(rnd + 1)))
    return x0 ^ x1


def _sample_kernel(x_ref, idx_ref, val_ref, *, ncols, bc):
    j = pl.program_id(0)
    x = x_ref[...]
    rows, cols = x.shape
    gcol = lax.broadcasted_iota(jnp.int32, x.shape, 1) + j * np.int32(bc)
    row = lax.broadcasted_iota(jnp.int32, x.shape, 0)
    i = row * np.int32(ncols) + gcol

    bits = _threefry_bits(i)
    fb = lax.shift_right_logical(bits, np.int32(9)) | np.int32(_EXP_ONE)
    f = lax.bitcast_convert_type(fb, jnp.float32) - np.float32(1.0)
    u = jnp.maximum(f, np.float32(_TINY))
    g = -jnp.log(-jnp.log(u))

    v = jnp.where(gcol < np.int32(ncols), x + g, -jnp.inf)
    bm = jnp.max(v, axis=1, keepdims=True)
    cand = jnp.where(v == bm, gcol, np.int32(_BIG_I32))
    bi = jnp.min(cand, axis=1, keepdims=True)

    @pl.when(j == 0)
    def _():
        val_ref[...] = bm
        idx_ref[...] = bi

    @pl.when(j != 0)
    def _():
        better = bm > val_ref[...]
        val_ref[...] = jnp.where(better, bm, val_ref[...])
        idx_ref[...] = jnp.where(better, bi, idx_ref[...])


@functools.partial(jax.jit, static_argnames=("block_cols",))
def _sample(outputs, block_cols=3072):
    rows, ncols = outputs.shape
    nblk = pl.cdiv(ncols, block_cols)
    idx, _ = pl.pallas_call(
        functools.partial(_sample_kernel, ncols=ncols, bc=block_cols),
        grid=(nblk,),
        in_specs=[pl.BlockSpec((rows, block_cols), lambda j: (0, j))],
        out_specs=[
            pl.BlockSpec((rows, 1), lambda j: (0, 0)),
            pl.BlockSpec((rows, 1), lambda j: (0, 0)),
        ],
        out_shape=[
            jax.ShapeDtypeStruct((rows, 1), jnp.int32),
            jax.ShapeDtypeStruct((rows, 1), jnp.float32),
        ],
    )(outputs)
    return idx


def kernel(outputs):
    return _sample(outputs)


# speedup vs baseline: 1.7251x; 1.0412x over previous
import functools

import numpy as np
import jax
import jax.numpy as jnp
from jax import lax
from jax.experimental import pallas as pl


def _i32(v) -> int:
    return int(np.uint32(v).view(np.int32))


_K1 = 0
_K2 = 42
_KS2 = _i32(np.uint32(_K1) ^ np.uint32(_K2) ^ np.uint32(0x1BD11BDA))
_ROTS = ((13, 15, 26, 6), (17, 29, 16, 24))
_KS = (_i32(_K1), _i32(_K2), _KS2)
_TINY = float(np.finfo(np.float32).tiny)
_EXP_ONE = _i32(0x3F800000)
_BIG_I32 = np.iinfo(np.int32).max


def _rotl(x, r):
    return lax.shift_left(x, np.int32(r)) | lax.shift_right_logical(
        x, np.int32(32 - r))


def _threefry_bits(i):
    x0 = jnp.full(i.shape, _KS[0], jnp.int32)
    x1 = i + np.int32(_KS[1])
    for rnd in range(5):
        for r in _ROTS[rnd % 2]:
            x0 = x0 + x1
            x1 = _rotl(x1, r)
            x1 = x1 ^ x0
        x0 = x0 + np.int32(_KS[(rnd + 1) % 3])
        x1 = x1 + np.int32(_i32(np.uint32(_KS[(rnd + 2) % 3]) + np.uint32(rnd + 1)))
    return x0 ^ x1


def _sample_kernel(x_ref, idx_ref, val_ref, *, ncols, bc, sub):
    j = pl.program_id(0)
    base_col = j * np.int32(bc)
    bm_acc = None
    bi_acc = None
    for s in range(bc // sub):
        x = x_ref[:, s * sub:(s + 1) * sub]
        gcol = (lax.broadcasted_iota(jnp.int32, x.shape, 1)
                + (base_col + np.int32(s * sub)))
        row = lax.broadcasted_iota(jnp.int32, x.shape, 0)
        i = row * np.int32(ncols) + gcol

        bits = _threefry_bits(i)
        fb = lax.shift_right_logical(bits, np.int32(9)) | np.int32(_EXP_ONE)
        f = lax.bitcast_convert_type(fb, jnp.float32) - np.float32(1.0)
        u = jnp.maximum(f, np.float32(_TINY))
        g = -jnp.log(-jnp.log(u))

        v = jnp.where(gcol < np.int32(ncols), x + g, -jnp.inf)
        bm = jnp.max(v, axis=1, keepdims=True)
        cand = jnp.where(v == bm, gcol, np.int32(_BIG_I32))
        bi = jnp.min(cand, axis=1, keepdims=True)
        if bm_acc is None:
            bm_acc, bi_acc = bm, bi
        else:
            better = bm > bm_acc
            bm_acc = jnp.where(better, bm, bm_acc)
            bi_acc = jnp.where(better, bi, bi_acc)

    @pl.when(j == 0)
    def _():
        val_ref[...] = bm_acc
        idx_ref[...] = bi_acc

    @pl.when(j != 0)
    def _():
        better = bm_acc > val_ref[...]
        val_ref[...] = jnp.where(better, bm_acc, val_ref[...])
        idx_ref[...] = jnp.where(better, bi_acc, idx_ref[...])


@functools.partial(jax.jit, static_argnames=("block_cols", "sub_cols"))
def _sample(outputs, block_cols=6144, sub_cols=1024):
    rows, ncols = outputs.shape
    nblk = pl.cdiv(ncols, block_cols)
    idx, _ = pl.pallas_call(
        functools.partial(_sample_kernel, ncols=ncols, bc=block_cols,
                          sub=sub_cols),
        grid=(nblk,),
        in_specs=[pl.BlockSpec((rows, block_cols), lambda j: (0, j))],
        out_specs=[
            pl.BlockSpec((rows, 1), lambda j: (0, 0)),
            pl.BlockSpec((rows, 1), lambda j: (0, 0)),
        ],
        out_shape=[
            jax.ShapeDtypeStruct((rows, 1), jnp.int32),
            jax.ShapeDtypeStruct((rows, 1), jnp.float32),
        ],
    )(outputs)
    return idx


def kernel(outputs):
    return _sample(outputs)


# speedup vs baseline: 1.7327x; 1.0044x over previous
import functools

import numpy as np
import jax
import jax.numpy as jnp
from jax import lax
from jax.experimental import pallas as pl


def _i32(v) -> int:
    return int(np.uint32(v).view(np.int32))


_K1 = 0
_K2 = 42
_KS2 = _i32(np.uint32(_K1) ^ np.uint32(_K2) ^ np.uint32(0x1BD11BDA))
_ROTS = ((13, 15, 26, 6), (17, 29, 16, 24))
_KS = (_i32(_K1), _i32(_K2), _KS2)
_TINY = float(np.finfo(np.float32).tiny)
_EXP_ONE = _i32(0x3F800000)
_BIG_I32 = np.iinfo(np.int32).max


def _rotl(x, r):
    return lax.shift_left(x, np.int32(r)) | lax.shift_right_logical(
        x, np.int32(32 - r))


def _threefry_bits(i):
    x0 = jnp.full(i.shape, _KS[0], jnp.int32)
    x1 = i + np.int32(_KS[1])
    for rnd in range(5):
        for r in _ROTS[rnd % 2]:
            x0 = x0 + x1
            x1 = _rotl(x1, r)
            x1 = x1 ^ x0
        x0 = x0 + np.int32(_KS[(rnd + 1) % 3])
        x1 = x1 + np.int32(_i32(np.uint32(_KS[(rnd + 2) % 3]) + np.uint32(rnd + 1)))
    return x0 ^ x1


def _sample_kernel(x_ref, idx_ref, val_ref, *, ncols, bc, sub):
    j = pl.program_id(0)
    base_col = j * np.int32(bc)
    bm_acc = None
    bi_acc = None
    for s in range(bc // sub):
        x = x_ref[:, s * sub:(s + 1) * sub]
        gcol = (lax.broadcasted_iota(jnp.int32, x.shape, 1)
                + (base_col + np.int32(s * sub)))
        row = lax.broadcasted_iota(jnp.int32, x.shape, 0)
        i = row * np.int32(ncols) + gcol

        bits = _threefry_bits(i)
        fb = lax.shift_right_logical(bits, np.int32(9)) | np.int32(_EXP_ONE)
        f = lax.bitcast_convert_type(fb, jnp.float32) - np.float32(1.0)
        u = jnp.maximum(f, np.float32(_TINY))
        g = -jnp.log(-jnp.log(u))

        v = jnp.where(gcol < np.int32(ncols), x + g, -jnp.inf)
        bm = jnp.max(v, axis=1, keepdims=True)
        cand = jnp.where(v == bm, gcol, np.int32(_BIG_I32))
        bi = jnp.min(cand, axis=1, keepdims=True)
        if bm_acc is None:
            bm_acc, bi_acc = bm, bi
        else:
            better = bm > bm_acc
            bm_acc = jnp.where(better, bm, bm_acc)
            bi_acc = jnp.where(better, bi, bi_acc)

    @pl.when(j == 0)
    def _():
        val_ref[...] = bm_acc
        idx_ref[...] = bi_acc

    @pl.when(j != 0)
    def _():
        better = bm_acc > val_ref[...]
        val_ref[...] = jnp.where(better, bm_acc, val_ref[...])
        idx_ref[...] = jnp.where(better, bi_acc, idx_ref[...])


@functools.partial(jax.jit, static_argnames=("block_cols", "sub_cols"))
def _sample(outputs, block_cols=12288, sub_cols=1024):
    rows, ncols = outputs.shape
    nblk = pl.cdiv(ncols, block_cols)
    idx, _ = pl.pallas_call(
        functools.partial(_sample_kernel, ncols=ncols, bc=block_cols,
                          sub=sub_cols),
        grid=(nblk,),
        in_specs=[pl.BlockSpec((rows, block_cols), lambda j: (0, j))],
        out_specs=[
            pl.BlockSpec((rows, 1), lambda j: (0, 0)),
            pl.BlockSpec((rows, 1), lambda j: (0, 0)),
        ],
        out_shape=[
            jax.ShapeDtypeStruct((rows, 1), jnp.int32),
            jax.ShapeDtypeStruct((rows, 1), jnp.float32),
        ],
    )(outputs)
    return idx


def kernel(outputs):
    return _sample(outputs)
